# baseline (speedup 1.0000x reference)
import functools

import jax
import jax.numpy as jnp
from jax import lax
from jax.experimental import pallas as pl
from jax.experimental.pallas import tpu as pltpu
from jax.experimental.pallas import tpu_sc as plsc

NUM_BEV = 64
NX, NY, NZ = 432, 496, 1
M = 40000
NCELL = NZ * NX * NY
MPAD = 40960
PAD_SLOT = M
TBL_W = 40016

NC = 2
NS = 16
L = 16
NTILES = NC * NS
CH_PER_TILE = NUM_BEV // NTILES
CELLS_PER_SUB = NCELL // NS
IDX_CHUNK = 2560
NIDX_CHUNK = MPAD // IDX_CHUNK
ROWS_B = 8
NCHUNK_B = NX // ROWS_B
INV_CHUNK = ROWS_B * NY
U_FILL = 9
U_SCAN = 8
NYV = NY // L


def _transpose_body(p_ref, o_ref):
    o_ref[...] = p_ref[...].T


def _idx_body(c_ref, o_ref):
    j = pl.program_id(0)
    c = c_ref[...]
    rows = lax.broadcasted_iota(jnp.int32, (2048,), 0) + j * 2048
    ok = (c[:, 0] == 0) & (c[:, 1] == 0) & (rows < M)
    idx_t = c[:, 3] * NY + c[:, 2]
    o_ref[...] = jnp.where(ok, idx_t, jnp.int32(1 << 22))


def _idx_kernel(coords):
    return pl.pallas_call(
        _idx_body,
        grid=(MPAD // 2048,),
        in_specs=[pl.BlockSpec((2048, 4), lambda j: (j, 0))],
        out_specs=pl.BlockSpec((2048,), lambda j: (j,)),
        out_shape=jax.ShapeDtypeStruct((MPAD,), jnp.int32),
    )(coords)


def _transpose(p):
    return pl.pallas_call(
        _transpose_body,
        grid=(MPAD // 2048,),
        in_specs=[pl.BlockSpec((2048, NUM_BEV), lambda j: (j, 0))],
        out_specs=pl.BlockSpec((NUM_BEV, 2048), lambda j: (0, j)),
        out_shape=jax.ShapeDtypeStruct((NUM_BEV, MPAD), jnp.float32),
    )(p)


def _sc_body(pt_hbm, idx_hbm, out_hbm,
             tbl0, tbl1, invbuf,
             idx0, idx1, ob0a, ob0b, ob1a, ob1b,
             spinv,
             sem_t0, sem_t1, sem_ix0, sem_ix1,
             sem_inv0, sem_inv1, sem_o0a, sem_o0b, sem_o1a, sem_o1b):
    core = lax.axis_index("c")
    sub = lax.axis_index("s")
    wid = core * NS + sub
    c0 = wid * CH_PER_TILE

    pltpu.make_async_copy(pt_hbm.at[pl.ds(c0 * MPAD, TBL_W)], tbl0,
                          sem_t0).start()
    pltpu.make_async_copy(pt_hbm.at[pl.ds((c0 + 1) * MPAD, TBL_W)], tbl1,
                          sem_t1).start()

    fill = jnp.full((L,), PAD_SLOT, jnp.int32)

    @plsc.parallel_loop(0, CELLS_PER_SUB // L, 1, unroll=8)
    def fill_step(i):
        invbuf[pl.ds(i * L, L)] = fill

    cell_base = sub * CELLS_PER_SUB
    iota = lax.iota(jnp.int32, L)
    idxbufs = (idx0, idx1)
    idxsems = (sem_ix0, sem_ix1)

    def idx_dma(c, s):
        return pltpu.make_async_copy(
            idx_hbm.at[pl.ds(c * IDX_CHUNK, IDX_CHUNK)], idxbufs[s],
            idxsems[s])

    def scan_slot(c, s):
        idx_dma(c, s).wait()
        buf = idxbufs[s]

        @plsc.parallel_loop(0, IDX_CHUNK // L, 1, unroll=8)
        def scan_step(k):
            off = k * L
            iv = buf[pl.ds(off, L)]
            rel = iv - cell_base
            msk = (rel >= 0) & (rel < CELLS_PER_SUB)
            relc = jnp.minimum(jnp.maximum(rel, 0), CELLS_PER_SUB - 1)
            vals = iota + (c * IDX_CHUNK + off)
            plsc.store_scatter(invbuf, [relc], vals, mask=msk)

    idx_dma(0, 0).start()

    def scan_pair(p, carry):
        c = p * 2

        @pl.when(c + 1 < NIDX_CHUNK)
        def _():
            idx_dma(c + 1, 1).start()

        scan_slot(c, 0)

        @pl.when(c + 2 < NIDX_CHUNK)
        def _():
            idx_dma(c + 2, 0).start()

        @pl.when(c + 1 < NIDX_CHUNK)
        def _():
            scan_slot(c + 1, 1)

        return carry

    lax.fori_loop(0, (NIDX_CHUNK + 1) // 2, scan_pair, 0)

    pltpu.sync_copy(invbuf, spinv.at[pl.ds(cell_base, CELLS_PER_SUB)])
    plsc.subcore_barrier()

    pltpu.make_async_copy(pt_hbm.at[pl.ds(c0 * MPAD, TBL_W)], tbl0,
                          sem_t0).wait()
    pltpu.make_async_copy(pt_hbm.at[pl.ds((c0 + 1) * MPAD, TBL_W)], tbl1,
                          sem_t1).wait()
    zpad = jnp.zeros((L,), jnp.float32)
    tbl0[pl.ds(PAD_SLOT, L)] = zpad
    tbl1[pl.ds(PAD_SLOT, L)] = zpad

    invsems = (sem_inv0, sem_inv1)
    obufs = ((ob0a, ob1a), (ob0b, ob1b))
    osems = ((sem_o0a, sem_o1a), (sem_o0b, sem_o1b))

    def inv_dma(c, s):
        return pltpu.make_async_copy(
            spinv.at[pl.ds(c * INV_CHUNK, INV_CHUNK)],
            invbuf.at[pl.ds(s * INV_CHUNK, INV_CHUNK)], invsems[s])

    def out_dma(ch, c, s):
        return pltpu.make_async_copy(
            obufs[s][ch],
            out_hbm.at[0, c0 + ch, pl.ds(c * ROWS_B, ROWS_B), :],
            osems[s][ch])

    def gather_slot(c, s, drain):
        inv_dma(c, s).wait()

        @pl.when(drain)
        def _():
            out_dma(0, 0, s).wait()
            out_dma(1, 0, s).wait()

        o0, o1 = obufs[s]

        @plsc.parallel_loop(0, NYV, 1, unroll=2)
        def g_step(u):
            for r in range(ROWS_B):
                src = pl.ds(s * INV_CHUNK + r * NY + u * L, L)
                iv = invbuf[src]
                o0[r, pl.ds(u * L, L)] = plsc.load_gather(tbl0, [iv])
                o1[r, pl.ds(u * L, L)] = plsc.load_gather(tbl1, [iv])
        out_dma(0, c, s).start()
        out_dma(1, c, s).start()

    inv_dma(0, 0).start()

    def gather_pair(p, carry):
        c = p * 2
        inv_dma(c + 1, 1).start()
        gather_slot(c, 0, p > 0)

        @pl.when(c + 2 < NCHUNK_B)
        def _():
            inv_dma(c + 2, 0).start()

        gather_slot(c + 1, 1, p > 0)
        return carry

    lax.fori_loop(0, NCHUNK_B // 2, gather_pair, 0)
    out_dma(0, 0, 0).wait()
    out_dma(1, 0, 0).wait()
    out_dma(0, 0, 1).wait()
    out_dma(1, 0, 1).wait()


@functools.partial(
    pl.kernel,
    mesh=plsc.VectorSubcoreMesh(core_axis_name="c", subcore_axis_name="s",
                                num_cores=NC, num_subcores=NS),
    out_type=jax.ShapeDtypeStruct((1, NUM_BEV, NX, NY), jnp.float32),
    compiler_params=pltpu.CompilerParams(needs_layout_passes=False),
    scratch_types=[
        pltpu.VMEM((TBL_W,), jnp.float32),
        pltpu.VMEM((TBL_W,), jnp.float32),
        pltpu.VMEM((CELLS_PER_SUB,), jnp.int32),
        pltpu.VMEM((IDX_CHUNK,), jnp.int32),
        pltpu.VMEM((IDX_CHUNK,), jnp.int32),
        pltpu.VMEM((ROWS_B, NY), jnp.float32),
        pltpu.VMEM((ROWS_B, NY), jnp.float32),
        pltpu.VMEM((ROWS_B, NY), jnp.float32),
        pltpu.VMEM((ROWS_B, NY), jnp.float32),
        pltpu.VMEM_SHARED((NCELL,), jnp.int32),
        pltpu.SemaphoreType.DMA,
        pltpu.SemaphoreType.DMA,
        pltpu.SemaphoreType.DMA,
        pltpu.SemaphoreType.DMA,
        pltpu.SemaphoreType.DMA,
        pltpu.SemaphoreType.DMA,
        pltpu.SemaphoreType.DMA,
        pltpu.SemaphoreType.DMA,
        pltpu.SemaphoreType.DMA,
        pltpu.SemaphoreType.DMA,
    ],
)
def _sc_scatter(pt_hbm, idx_hbm, out_hbm, *rest):
    _sc_body(pt_hbm, idx_hbm, out_hbm, *rest)


@jax.jit
def kernel(pillar_features, coords):
    coords = coords.astype(jnp.int32)
    idx_pad = _idx_kernel(coords)
    ptt = jnp.zeros((NUM_BEV, MPAD), jnp.float32)
    ptt = lax.dynamic_update_slice(ptt, pillar_features.T, (0, 0))
    pt = ptt.reshape(-1)
    out_t = _sc_scatter(pt, idx_pad)
    return jnp.swapaxes(out_t, 2, 3)


# speedup vs baseline: 1.4100x; 1.4100x over previous
import functools

import jax
import jax.numpy as jnp
from jax import lax
from jax.experimental import pallas as pl
from jax.experimental.pallas import tpu as pltpu
from jax.experimental.pallas import tpu_sc as plsc

NUM_BEV = 64
NX, NY, NZ = 432, 496, 1
M = 40000
NCELL = NZ * NX * NY
MPAD = 40960
PAD_SLOT = M
TBL_W = 40016

NC = 2
NS = 16
L = 16
NTILES = NC * NS
CH_PER_TILE = NUM_BEV // NTILES
CELLS_PER_SUB = NCELL // NS
IDX_CHUNK = 2560
NIDX_CHUNK = MPAD // IDX_CHUNK
ROWS_B = 8
NCHUNK_B = NX // ROWS_B
INV_CHUNK = ROWS_B * NY
U_FILL = 9
U_SCAN = 8
NYV = NY // L


def _transpose_body(p_ref, o_ref):
    o_ref[...] = p_ref[...].T


def _transpose(p):
    return pl.pallas_call(
        _transpose_body,
        grid=(MPAD // 2048,),
        in_specs=[pl.BlockSpec((2048, NUM_BEV), lambda j: (j, 0))],
        out_specs=pl.BlockSpec((NUM_BEV, 2048), lambda j: (0, j)),
        out_shape=jax.ShapeDtypeStruct((NUM_BEV, MPAD), jnp.float32),
    )(p)


def _sc_body(pt_hbm, idx_hbm, out_hbm,
             tbl0, tbl1, invbuf,
             idx0, idx1, ob0a, ob0b, ob1a, ob1b,
             spinv,
             sem_t0, sem_t1, sem_ix0, sem_ix1,
             sem_inv0, sem_inv1, sem_o0a, sem_o0b, sem_o1a, sem_o1b):
    core = lax.axis_index("c")
    sub = lax.axis_index("s")
    wid = core * NS + sub
    c0 = wid * CH_PER_TILE

    pltpu.make_async_copy(pt_hbm.at[pl.ds(c0 * MPAD, TBL_W)], tbl0,
                          sem_t0).start()
    pltpu.make_async_copy(pt_hbm.at[pl.ds((c0 + 1) * MPAD, TBL_W)], tbl1,
                          sem_t1).start()

    fill = jnp.full((L,), PAD_SLOT, jnp.int32)

    @plsc.parallel_loop(0, CELLS_PER_SUB // L, 1, unroll=8)
    def fill_step(i):
        invbuf[pl.ds(i * L, L)] = fill

    cell_base = sub * CELLS_PER_SUB
    iota = lax.iota(jnp.int32, L)
    idxbufs = (idx0, idx1)
    idxsems = (sem_ix0, sem_ix1)

    def idx_dma(c, s):
        return pltpu.make_async_copy(
            idx_hbm.at[pl.ds(c * IDX_CHUNK, IDX_CHUNK)], idxbufs[s],
            idxsems[s])

    def scan_slot(c, s):
        idx_dma(c, s).wait()
        buf = idxbufs[s]

        @plsc.parallel_loop(0, IDX_CHUNK // L, 1, unroll=8)
        def scan_step(k):
            off = k * L
            iv = buf[pl.ds(off, L)]
            rel = iv - cell_base
            msk = (rel >= 0) & (rel < CELLS_PER_SUB)
            relc = jnp.minimum(jnp.maximum(rel, 0), CELLS_PER_SUB - 1)
            vals = iota + (c * IDX_CHUNK + off)
            plsc.store_scatter(invbuf, [relc], vals, mask=msk)

    idx_dma(0, 0).start()

    def scan_pair(p, carry):
        c = p * 2

        @pl.when(c + 1 < NIDX_CHUNK)
        def _():
            idx_dma(c + 1, 1).start()

        scan_slot(c, 0)

        @pl.when(c + 2 < NIDX_CHUNK)
        def _():
            idx_dma(c + 2, 0).start()

        @pl.when(c + 1 < NIDX_CHUNK)
        def _():
            scan_slot(c + 1, 1)

        return carry

    lax.fori_loop(0, (NIDX_CHUNK + 1) // 2, scan_pair, 0)

    pltpu.sync_copy(invbuf, spinv.at[pl.ds(cell_base, CELLS_PER_SUB)])
    plsc.subcore_barrier()

    pltpu.make_async_copy(pt_hbm.at[pl.ds(c0 * MPAD, TBL_W)], tbl0,
                          sem_t0).wait()
    pltpu.make_async_copy(pt_hbm.at[pl.ds((c0 + 1) * MPAD, TBL_W)], tbl1,
                          sem_t1).wait()
    zpad = jnp.zeros((L,), jnp.float32)
    tbl0[pl.ds(PAD_SLOT, L)] = zpad
    tbl1[pl.ds(PAD_SLOT, L)] = zpad

    invsems = (sem_inv0, sem_inv1)
    obufs = ((ob0a, ob1a), (ob0b, ob1b))
    osems = ((sem_o0a, sem_o1a), (sem_o0b, sem_o1b))

    def inv_dma(c, s):
        return pltpu.make_async_copy(
            spinv.at[pl.ds(c * INV_CHUNK, INV_CHUNK)],
            invbuf.at[pl.ds(s * INV_CHUNK, INV_CHUNK)], invsems[s])

    def out_dma(ch, c, s):
        return pltpu.make_async_copy(
            obufs[s][ch],
            out_hbm.at[0, c0 + ch, pl.ds(c * ROWS_B, ROWS_B), :],
            osems[s][ch])

    def gather_slot(c, s, drain):
        inv_dma(c, s).wait()

        @pl.when(drain)
        def _():
            out_dma(0, 0, s).wait()
            out_dma(1, 0, s).wait()

        o0, o1 = obufs[s]

        @plsc.parallel_loop(0, NYV, 1, unroll=2)
        def g_step(u):
            for r in range(ROWS_B):
                src = pl.ds(s * INV_CHUNK + r * NY + u * L, L)
                iv = invbuf[src]
                o0[r, pl.ds(u * L, L)] = plsc.load_gather(tbl0, [iv])
                o1[r, pl.ds(u * L, L)] = plsc.load_gather(tbl1, [iv])
        out_dma(0, c, s).start()
        out_dma(1, c, s).start()

    inv_dma(0, 0).start()

    def gather_pair(p, carry):
        c = p * 2
        inv_dma(c + 1, 1).start()
        gather_slot(c, 0, p > 0)

        @pl.when(c + 2 < NCHUNK_B)
        def _():
            inv_dma(c + 2, 0).start()

        gather_slot(c + 1, 1, p > 0)
        return carry

    lax.fori_loop(0, NCHUNK_B // 2, gather_pair, 0)
    out_dma(0, 0, 0).wait()
    out_dma(1, 0, 0).wait()
    out_dma(0, 0, 1).wait()
    out_dma(1, 0, 1).wait()


@functools.partial(
    pl.kernel,
    mesh=plsc.VectorSubcoreMesh(core_axis_name="c", subcore_axis_name="s",
                                num_cores=NC, num_subcores=NS),
    out_type=jax.ShapeDtypeStruct((1, NUM_BEV, NX, NY), jnp.float32),
    compiler_params=pltpu.CompilerParams(needs_layout_passes=False),
    scratch_types=[
        pltpu.VMEM((TBL_W,), jnp.float32),
        pltpu.VMEM((TBL_W,), jnp.float32),
        pltpu.VMEM((CELLS_PER_SUB,), jnp.int32),
        pltpu.VMEM((IDX_CHUNK,), jnp.int32),
        pltpu.VMEM((IDX_CHUNK,), jnp.int32),
        pltpu.VMEM((ROWS_B, NY), jnp.float32),
        pltpu.VMEM((ROWS_B, NY), jnp.float32),
        pltpu.VMEM((ROWS_B, NY), jnp.float32),
        pltpu.VMEM((ROWS_B, NY), jnp.float32),
        pltpu.VMEM_SHARED((NCELL,), jnp.int32),
        pltpu.SemaphoreType.DMA,
        pltpu.SemaphoreType.DMA,
        pltpu.SemaphoreType.DMA,
        pltpu.SemaphoreType.DMA,
        pltpu.SemaphoreType.DMA,
        pltpu.SemaphoreType.DMA,
        pltpu.SemaphoreType.DMA,
        pltpu.SemaphoreType.DMA,
        pltpu.SemaphoreType.DMA,
        pltpu.SemaphoreType.DMA,
    ],
)
def _sc_scatter(pt_hbm, idx_hbm, out_hbm, *rest):
    _sc_body(pt_hbm, idx_hbm, out_hbm, *rest)


@jax.jit
def kernel(pillar_features, coords):
    coords = coords.astype(jnp.int32)
    zyx_ok = (coords[:, 0] == 0) & (coords[:, 1] == 0)
    idx_t = coords[:, 3] * NY + coords[:, 2]
    idx_t = jnp.where(zyx_ok, idx_t, jnp.int32(-1))
    idx_pad = jnp.full((MPAD,), jnp.int32(1 << 22)).at[:M].set(idx_t)
    ptt = jnp.zeros((NUM_BEV, MPAD), jnp.float32)
    ptt = lax.dynamic_update_slice(ptt, pillar_features.T, (0, 0))
    pt = ptt.reshape(-1)
    out_t = _sc_scatter(pt, idx_pad)
    return jnp.swapaxes(out_t, 2, 3)


# speedup vs baseline: 1.4104x; 1.0003x over previous
import functools

import jax
import jax.numpy as jnp
from jax import lax
from jax.experimental import pallas as pl
from jax.experimental.pallas import tpu as pltpu
from jax.experimental.pallas import tpu_sc as plsc

NUM_BEV = 64
NX, NY, NZ = 432, 496, 1
M = 40000
NCELL = NZ * NX * NY
MPAD = 40960
PAD_SLOT = M
TBL_W = 40016

NC = 2
NS = 16
L = 16
NTILES = NC * NS
CH_PER_TILE = NUM_BEV // NTILES
CELLS_PER_SUB = NCELL // NS
IDX_CHUNK = 2560
NIDX_CHUNK = MPAD // IDX_CHUNK
ROWS_B = 8
NCHUNK_B = NX // ROWS_B
INV_CHUNK = ROWS_B * NY
NYV = NY // L


def _sc_body(pt_hbm, idx_hbm, out_hbm,
             tbl0, tbl1, invbuf,
             idx0, idx1, ob0a, ob0b, ob1a, ob1b,
             spinv,
             sem_t0, sem_t1, sem_ix0, sem_ix1,
             sem_inv0, sem_inv1, sem_o0a, sem_o0b, sem_o1a, sem_o1b):
    core = lax.axis_index("c")
    sub = lax.axis_index("s")
    wid = core * NS + sub
    c0 = wid * CH_PER_TILE

    pltpu.make_async_copy(pt_hbm.at[pl.ds(c0 * MPAD, TBL_W)], tbl0,
                          sem_t0).start()
    pltpu.make_async_copy(pt_hbm.at[pl.ds((c0 + 1) * MPAD, TBL_W)], tbl1,
                          sem_t1).start()

    fill = jnp.full((L,), PAD_SLOT, jnp.int32)

    @plsc.parallel_loop(0, CELLS_PER_SUB // L, 1, unroll=8)
    def fill_step(i):
        invbuf[pl.ds(i * L, L)] = fill

    cell_base = sub * CELLS_PER_SUB
    iota = lax.iota(jnp.int32, L)
    idxbufs = (idx0, idx1)
    idxsems = (sem_ix0, sem_ix1)

    def idx_dma(c, s):
        return pltpu.make_async_copy(
            idx_hbm.at[pl.ds(c * IDX_CHUNK, IDX_CHUNK)], idxbufs[s],
            idxsems[s])

    def scan_slot(c, s):
        idx_dma(c, s).wait()
        buf = idxbufs[s]

        @plsc.parallel_loop(0, IDX_CHUNK // L, 1, unroll=8)
        def scan_step(k):
            off = k * L
            iv = buf[pl.ds(off, L)]
            rel = iv - cell_base
            msk = (rel >= 0) & (rel < CELLS_PER_SUB)
            relc = jnp.minimum(jnp.maximum(rel, 0), CELLS_PER_SUB - 1)
            vals = iota + (c * IDX_CHUNK + off)
            plsc.store_scatter(invbuf, [relc], vals, mask=msk)

    idx_dma(0, 0).start()

    def scan_pair(p, carry):
        c = p * 2

        @pl.when(c + 1 < NIDX_CHUNK)
        def _():
            idx_dma(c + 1, 1).start()

        scan_slot(c, 0)

        @pl.when(c + 2 < NIDX_CHUNK)
        def _():
            idx_dma(c + 2, 0).start()

        @pl.when(c + 1 < NIDX_CHUNK)
        def _():
            scan_slot(c + 1, 1)

        return carry

    lax.fori_loop(0, (NIDX_CHUNK + 1) // 2, scan_pair, 0)

    pltpu.sync_copy(invbuf, spinv.at[pl.ds(cell_base, CELLS_PER_SUB)])
    plsc.subcore_barrier()

    pltpu.make_async_copy(pt_hbm.at[pl.ds(c0 * MPAD, TBL_W)], tbl0,
                          sem_t0).wait()
    pltpu.make_async_copy(pt_hbm.at[pl.ds((c0 + 1) * MPAD, TBL_W)], tbl1,
                          sem_t1).wait()
    zpad = jnp.zeros((L,), jnp.float32)
    tbl0[pl.ds(PAD_SLOT, L)] = zpad
    tbl1[pl.ds(PAD_SLOT, L)] = zpad

    invsems = (sem_inv0, sem_inv1)
    obufs = ((ob0a, ob1a), (ob0b, ob1b))
    osems = ((sem_o0a, sem_o1a), (sem_o0b, sem_o1b))

    def inv_dma(c, s):
        return pltpu.make_async_copy(
            spinv.at[pl.ds(c * INV_CHUNK, INV_CHUNK)],
            invbuf.at[pl.ds(s * INV_CHUNK, INV_CHUNK)], invsems[s])

    def out_dma(ch, c, s):
        return pltpu.make_async_copy(
            obufs[s][ch],
            out_hbm.at[0, c0 + ch, pl.ds(c * ROWS_B, ROWS_B), :],
            osems[s][ch])

    def gather_slot(c, s, drain):
        inv_dma(c, s).wait()

        @pl.when(drain)
        def _():
            out_dma(0, 0, s).wait()
            out_dma(1, 0, s).wait()

        o0, o1 = obufs[s]

        @plsc.parallel_loop(0, NYV, 1, unroll=2)
        def g_step(u):
            for r in range(ROWS_B):
                src = pl.ds(s * INV_CHUNK + r * NY + u * L, L)
                iv = invbuf[src]
                o0[r, pl.ds(u * L, L)] = plsc.load_gather(tbl0, [iv])
                o1[r, pl.ds(u * L, L)] = plsc.load_gather(tbl1, [iv])
        out_dma(0, c, s).start()
        out_dma(1, c, s).start()

    inv_dma(0, 0).start()

    def gather_pair(p, carry):
        c = p * 2
        inv_dma(c + 1, 1).start()
        gather_slot(c, 0, p > 0)

        @pl.when(c + 2 < NCHUNK_B)
        def _():
            inv_dma(c + 2, 0).start()

        gather_slot(c + 1, 1, p > 0)
        return carry

    lax.fori_loop(0, NCHUNK_B // 2, gather_pair, 0)
    out_dma(0, 0, 0).wait()
    out_dma(1, 0, 0).wait()
    out_dma(0, 0, 1).wait()
    out_dma(1, 0, 1).wait()


@functools.partial(
    pl.kernel,
    mesh=plsc.VectorSubcoreMesh(core_axis_name="c", subcore_axis_name="s",
                                num_cores=NC, num_subcores=NS),
    out_type=jax.ShapeDtypeStruct((1, NUM_BEV, NX, NY), jnp.float32),
    compiler_params=pltpu.CompilerParams(needs_layout_passes=False),
    scratch_types=[
        pltpu.VMEM((TBL_W,), jnp.float32),
        pltpu.VMEM((TBL_W,), jnp.float32),
        pltpu.VMEM((CELLS_PER_SUB,), jnp.int32),
        pltpu.VMEM((IDX_CHUNK,), jnp.int32),
        pltpu.VMEM((IDX_CHUNK,), jnp.int32),
        pltpu.VMEM((ROWS_B, NY), jnp.float32),
        pltpu.VMEM((ROWS_B, NY), jnp.float32),
        pltpu.VMEM((ROWS_B, NY), jnp.float32),
        pltpu.VMEM((ROWS_B, NY), jnp.float32),
        pltpu.VMEM_SHARED((NCELL,), jnp.int32),
        pltpu.SemaphoreType.DMA,
        pltpu.SemaphoreType.DMA,
        pltpu.SemaphoreType.DMA,
        pltpu.SemaphoreType.DMA,
        pltpu.SemaphoreType.DMA,
        pltpu.SemaphoreType.DMA,
        pltpu.SemaphoreType.DMA,
        pltpu.SemaphoreType.DMA,
        pltpu.SemaphoreType.DMA,
        pltpu.SemaphoreType.DMA,
    ],
)
def _sc_scatter(pt_hbm, idx_hbm, out_hbm, *rest):
    _sc_body(pt_hbm, idx_hbm, out_hbm, *rest)


@jax.jit
def kernel(pillar_features, coords):
    coords = coords.astype(jnp.int32)
    zyx_ok = (coords[:, 0] == 0) & (coords[:, 1] == 0)
    idx_t = coords[:, 3] * NY + coords[:, 2]
    idx_t = jnp.where(zyx_ok, idx_t, jnp.int32(-1))
    idx_pad = jnp.full((MPAD,), jnp.int32(1 << 22)).at[:M].set(idx_t)
    ptt = jnp.zeros((NUM_BEV, MPAD), jnp.float32)
    ptt = lax.dynamic_update_slice(ptt, pillar_features.T, (0, 0))
    pt = ptt.reshape(-1)
    out_t = _sc_scatter(pt, idx_pad)
    return jnp.swapaxes(out_t, 2, 3)


# speedup vs baseline: 1.5599x; 1.1060x over previous
import functools

import jax
import jax.numpy as jnp
from jax import lax
from jax.experimental import pallas as pl
from jax.experimental.pallas import tpu as pltpu
from jax.experimental.pallas import tpu_sc as plsc

NUM_BEV = 64
NX, NY, NZ = 432, 496, 1
M = 40000
NCELL = NZ * NX * NY
MPAD = 40960
PAD_SLOT = M
TBL_W = 40016

NC = 2
NS = 16
L = 16
NTILES = NC * NS
CH_PER_TILE = NUM_BEV // NTILES
CELLS_PER_SUB = NCELL // NS
IDX_CHUNK = 2560
FILL_Q = 4464
ROWS_B = 8
NCHUNK_B = NX // ROWS_B
INV_CHUNK = ROWS_B * NY
NYV = NY // L


def _sc_body(pt_hbm, idx_hbm, out_hbm,
             tbl0, tbl1, irdbuf,
             idx0, idx1, fbuf, ob0a, ob0b, ob1a, ob1b,
             spinv,
             sem_t0, sem_t1, sem_ix0, sem_ix1,
             sem_inv0, sem_inv1, sem_o0a, sem_o0b, sem_o1a, sem_o1b):
    core = lax.axis_index("c")
    sub = lax.axis_index("s")
    wid = core * NS + sub
    c0 = wid * CH_PER_TILE

    pltpu.make_async_copy(pt_hbm.at[pl.ds(c0 * MPAD, TBL_W)], tbl0,
                          sem_t0).start()
    pltpu.make_async_copy(pt_hbm.at[pl.ds((c0 + 1) * MPAD, TBL_W)], tbl1,
                          sem_t1).start()

    cell_base = sub * CELLS_PER_SUB
    iota = lax.iota(jnp.int32, L)
    fill = jnp.full((L,), PAD_SLOT, jnp.int32)

    @plsc.parallel_loop(0, FILL_Q // L, 1, unroll=8)
    def fill_step(i):
        fbuf[pl.ds(i * L, L)] = fill

    idx_cp = pltpu.make_async_copy(
        idx_hbm.at[pl.ds(sub * IDX_CHUNK, IDX_CHUNK)], idx0, sem_ix0)
    idx_cp.start()

    val_base = sub * IDX_CHUNK

    @plsc.parallel_loop(0, IDX_CHUNK // L, 1, unroll=8)
    def val_step(k):
        idx1[pl.ds(k * L, L)] = iota + (val_base + k * L)

    for q in range(3):
        pltpu.sync_copy(fbuf, spinv.at[pl.ds(cell_base + q * FILL_Q, FILL_Q)])
    idx_cp.wait()
    plsc.subcore_barrier()
    pltpu.sync_copy(idx1, spinv.at[idx0])
    plsc.subcore_barrier()

    pltpu.make_async_copy(pt_hbm.at[pl.ds(c0 * MPAD, TBL_W)], tbl0,
                          sem_t0).wait()
    pltpu.make_async_copy(pt_hbm.at[pl.ds((c0 + 1) * MPAD, TBL_W)], tbl1,
                          sem_t1).wait()
    zpad = jnp.zeros((L,), jnp.float32)
    tbl0[pl.ds(PAD_SLOT, L)] = zpad
    tbl1[pl.ds(PAD_SLOT, L)] = zpad

    invsems = (sem_inv0, sem_inv1)
    obufs = ((ob0a, ob1a), (ob0b, ob1b))
    osems = ((sem_o0a, sem_o1a), (sem_o0b, sem_o1b))

    def inv_dma(c, s):
        return pltpu.make_async_copy(
            spinv.at[pl.ds(c * INV_CHUNK, INV_CHUNK)],
            irdbuf.at[pl.ds(s * INV_CHUNK, INV_CHUNK)], invsems[s])

    def out_dma(ch, c, s):
        return pltpu.make_async_copy(
            obufs[s][ch],
            out_hbm.at[0, c0 + ch, pl.ds(c * ROWS_B, ROWS_B), :],
            osems[s][ch])

    def gather_slot(c, s, drain):
        inv_dma(c, s).wait()

        @pl.when(drain)
        def _():
            out_dma(0, 0, s).wait()
            out_dma(1, 0, s).wait()

        o0, o1 = obufs[s]

        @plsc.parallel_loop(0, NYV, 1, unroll=2)
        def g_step(u):
            for r in range(ROWS_B):
                src = pl.ds(s * INV_CHUNK + r * NY + u * L, L)
                iv = irdbuf[src]
                o0[r, pl.ds(u * L, L)] = plsc.load_gather(tbl0, [iv])
                o1[r, pl.ds(u * L, L)] = plsc.load_gather(tbl1, [iv])
        out_dma(0, c, s).start()
        out_dma(1, c, s).start()

    inv_dma(0, 0).start()

    def gather_pair(p, carry):
        c = p * 2
        inv_dma(c + 1, 1).start()
        gather_slot(c, 0, p > 0)

        @pl.when(c + 2 < NCHUNK_B)
        def _():
            inv_dma(c + 2, 0).start()

        gather_slot(c + 1, 1, p > 0)
        return carry

    lax.fori_loop(0, NCHUNK_B // 2, gather_pair, 0)
    out_dma(0, 0, 0).wait()
    out_dma(1, 0, 0).wait()
    out_dma(0, 0, 1).wait()
    out_dma(1, 0, 1).wait()


@functools.partial(
    pl.kernel,
    mesh=plsc.VectorSubcoreMesh(core_axis_name="c", subcore_axis_name="s",
                                num_cores=NC, num_subcores=NS),
    out_type=jax.ShapeDtypeStruct((1, NUM_BEV, NX, NY), jnp.float32),
    compiler_params=pltpu.CompilerParams(needs_layout_passes=False),
    scratch_types=[
        pltpu.VMEM((TBL_W,), jnp.float32),
        pltpu.VMEM((TBL_W,), jnp.float32),
        pltpu.VMEM((2 * INV_CHUNK,), jnp.int32),
        pltpu.VMEM((IDX_CHUNK,), jnp.int32),
        pltpu.VMEM((IDX_CHUNK,), jnp.int32),
        pltpu.VMEM((FILL_Q,), jnp.int32),
        pltpu.VMEM((ROWS_B, NY), jnp.float32),
        pltpu.VMEM((ROWS_B, NY), jnp.float32),
        pltpu.VMEM((ROWS_B, NY), jnp.float32),
        pltpu.VMEM((ROWS_B, NY), jnp.float32),
        pltpu.VMEM_SHARED((NCELL + 8,), jnp.int32),
        pltpu.SemaphoreType.DMA,
        pltpu.SemaphoreType.DMA,
        pltpu.SemaphoreType.DMA,
        pltpu.SemaphoreType.DMA,
        pltpu.SemaphoreType.DMA,
        pltpu.SemaphoreType.DMA,
        pltpu.SemaphoreType.DMA,
        pltpu.SemaphoreType.DMA,
        pltpu.SemaphoreType.DMA,
        pltpu.SemaphoreType.DMA,
    ],
)
def _sc_scatter(pt_hbm, idx_hbm, out_hbm, *rest):
    _sc_body(pt_hbm, idx_hbm, out_hbm, *rest)


@jax.jit
def kernel(pillar_features, coords):
    coords = coords.astype(jnp.int32)
    zyx_ok = (coords[:, 0] == 0) & (coords[:, 1] == 0)
    idx_t = coords[:, 3] * NY + coords[:, 2]
    idx_t = jnp.where(zyx_ok, idx_t, jnp.int32(-1))
    idx_t = jnp.where((idx_t >= 0) & (idx_t < NCELL), idx_t,
                      jnp.int32(NCELL))
    idx_pad = jnp.full((MPAD,), jnp.int32(NCELL)).at[:M].set(idx_t)
    ptt = jnp.zeros((NUM_BEV, MPAD), jnp.float32)
    ptt = lax.dynamic_update_slice(ptt, pillar_features.T, (0, 0))
    pt = ptt.reshape(-1)
    out_t = _sc_scatter(pt, idx_pad)
    return jnp.swapaxes(out_t, 2, 3)
